# baseline (speedup 1.0000x reference)
import jax
import jax.numpy as jnp
from jax.experimental import pallas as pl
from jax.experimental.pallas import tpu as pltpu
from jax.experimental.pallas import tpu_sc as plsc

_BLOCK = (40, 128)
_LANES = 16


def kernel(x, edge_index):
    n, d = x.shape
    mesh = plsc.VectorSubcoreMesh(core_axis_name="core", subcore_axis_name="subcore")

    @pl.kernel(out_type=jax.ShapeDtypeStruct((n, d), x.dtype), mesh=mesh)
    def _sc_double(x_hbm, o_hbm):
        def body(in_spmem, out_spmem):
            @pl.loop(0, _BLOCK[0])
            def _(r):
                @pl.loop(0, _BLOCK[1], step=_LANES)
                def _(c):
                    slc = (pl.ds(r, 1), pl.ds(c, _LANES))
                    v = in_spmem.at[*slc][...]
                    out_spmem.at[*slc][...] = v + v

        pltpu.emit_pipeline(
            body,
            grid=(n // _BLOCK[0], d // _BLOCK[1]),
            in_specs=[pl.BlockSpec(block_shape=_BLOCK, index_map=lambda i, j: (i, j))],
            out_specs=[pl.BlockSpec(block_shape=_BLOCK, index_map=lambda i, j: (i, j))],
            core_axis_name=("core", "subcore"),
            dimension_semantics=(pltpu.PARALLEL, pltpu.PARALLEL),
        )(x_hbm, o_hbm)

    return _sc_double(x)


# speedup vs baseline: 8.5705x; 8.5705x over previous
import jax
import jax.numpy as jnp
from jax.experimental import pallas as pl
from jax.experimental.pallas import tpu as pltpu

_CHUNK_ROWS = (2000, 2000, 2000, 2000, 2000)
_N_CHUNKS = len(_CHUNK_ROWS)
_OFFS = tuple(sum(_CHUNK_ROWS[:i]) for i in range(_N_CHUNKS))
_MAX_ROWS = max(_CHUNK_ROWS)


def _double_stream(x_hbm, o_hbm, xb, yb, in_sems, out_sems):
    for i in range(_N_CHUNKS):
        pltpu.make_async_copy(
            x_hbm.at[pl.ds(_OFFS[i], _CHUNK_ROWS[i]), :],
            xb.at[i, pl.ds(0, _CHUNK_ROWS[i])],
            in_sems.at[i],
        ).start()
    for i in range(_N_CHUNKS):
        pltpu.make_async_copy(
            x_hbm.at[pl.ds(_OFFS[i], _CHUNK_ROWS[i]), :],
            xb.at[i, pl.ds(0, _CHUNK_ROWS[i])],
            in_sems.at[i],
        ).wait()
        yb[i] = xb[i] + xb[i]
        pltpu.make_async_copy(
            yb.at[i, pl.ds(0, _CHUNK_ROWS[i])],
            o_hbm.at[pl.ds(_OFFS[i], _CHUNK_ROWS[i]), :],
            out_sems.at[i],
        ).start()
    for i in range(_N_CHUNKS):
        pltpu.make_async_copy(
            yb.at[i, pl.ds(0, _CHUNK_ROWS[i])],
            o_hbm.at[pl.ds(_OFFS[i], _CHUNK_ROWS[i]), :],
            out_sems.at[i],
        ).wait()


def kernel(x, edge_index):
    n, d = x.shape
    return pl.pallas_call(
        _double_stream,
        in_specs=[pl.BlockSpec(memory_space=pltpu.MemorySpace.HBM)],
        out_specs=pl.BlockSpec(memory_space=pltpu.MemorySpace.HBM),
        out_shape=jax.ShapeDtypeStruct((n, d), x.dtype),
        scratch_shapes=[
            pltpu.VMEM((_N_CHUNKS, _MAX_ROWS, d), x.dtype),
            pltpu.VMEM((_N_CHUNKS, _MAX_ROWS, d), x.dtype),
            pltpu.SemaphoreType.DMA((_N_CHUNKS,)),
            pltpu.SemaphoreType.DMA((_N_CHUNKS,)),
        ],
    )(x)
